# baseline (device time: 282924 ns/iter reference)
import jax
import jax.numpy as jnp
from jax import lax
from jax.experimental import pallas as pl
from jax.experimental.pallas import tpu as pltpu

N = 16
B = 64
D = 512
H = 1024


def kernel(x, Win0, Wout0, Win1, Wout1, Win2, Wout2):
    def body(x_ref, win0_ref, wout0_ref, win1_ref, wout1_ref, win2_ref,
             wout2_ref, out_ref, xfull, ypart, rsbuf, ag_ss, ag_rs,
             rs_ss, rs_rs):
        i = lax.axis_index("i")
        left = (i - 1 + N) % N
        right = (i + 1) % N

        barrier = pltpu.get_barrier_semaphore()
        for nbr in (left, right):
            pl.semaphore_signal(barrier, inc=1, device_id=(nbr,),
                                device_id_type=pl.DeviceIdType.MESH)
        pl.semaphore_wait(barrier, 2)

        def ring_ag():
            for h in range(N - 1):
                src_o = (i - h + N) % N
                rdma = pltpu.make_async_remote_copy(
                    src_ref=xfull.at[pl.ds(src_o * B, B)],
                    dst_ref=xfull.at[pl.ds(src_o * B, B)],
                    send_sem=ag_ss.at[h],
                    recv_sem=ag_rs.at[h],
                    device_id=(right,),
                    device_id_type=pl.DeviceIdType.MESH,
                )
                rdma.start()
                rdma.wait()

        def layer(win_ref, wout_ref):
            xw = xfull[...]
            h1 = jnp.maximum(
                jnp.dot(xw, win_ref[...].astype(jnp.bfloat16),
                        preferred_element_type=jnp.float32),
                0.0,
            ).astype(jnp.bfloat16)
            ypart[...] = jnp.dot(h1, wout_ref[...].astype(jnp.bfloat16),
                                 preferred_element_type=jnp.float32)

        def ring_rs():
            for s in range(N - 1):
                so = (i - s - 1 + N) % N
                ro = (i - s - 2 + 2 * N) % N
                rdma = pltpu.make_async_remote_copy(
                    src_ref=ypart.at[pl.ds(so * B, B)],
                    dst_ref=rsbuf.at[s],
                    send_sem=rs_ss.at[s],
                    recv_sem=rs_rs.at[s],
                    device_id=(right,),
                    device_id_type=pl.DeviceIdType.MESH,
                )
                rdma.start()
                rdma.wait()
                ypart[pl.ds(ro * B, B), :] = (
                    ypart[pl.ds(ro * B, B), :] + rsbuf[s]
                )

        xfull[pl.ds(i * B, B), :] = x_ref[...].astype(jnp.bfloat16)
        ring_ag()

        for li, (win_ref, wout_ref) in enumerate(
            [(win0_ref, wout0_ref), (win1_ref, wout1_ref),
             (win2_ref, wout2_ref)]
        ):
            layer(win_ref, wout_ref)
            ring_rs()
            if li < 2:
                xfull[pl.ds(i * B, B), :] = (
                    ypart[pl.ds(i * B, B), :].astype(jnp.bfloat16)
                )
                ring_ag()

        out_ref[...] = ypart[pl.ds(i * B, B), :]

    vmem = pl.BlockSpec(memory_space=pltpu.VMEM)
    return pl.pallas_call(
        body,
        out_shape=jax.ShapeDtypeStruct((B, D), jnp.float32),
        in_specs=[vmem] * 7,
        out_specs=vmem,
        scratch_shapes=[
            pltpu.VMEM((N * B, D), jnp.bfloat16),
            pltpu.VMEM((N * B, D), jnp.float32),
            pltpu.VMEM((N - 1, B, D), jnp.float32),
            pltpu.SemaphoreType.DMA((N - 1,)),
            pltpu.SemaphoreType.DMA((N - 1,)),
            pltpu.SemaphoreType.DMA((N - 1,)),
            pltpu.SemaphoreType.DMA((N - 1,)),
        ],
        compiler_params=pltpu.CompilerParams(collective_id=0),
    )(x, Win0, Wout0, Win1, Wout1, Win2, Wout2)


# device time: 159011 ns/iter; 1.7793x vs baseline; 1.7793x over previous
import jax
import jax.numpy as jnp
from jax import lax
from jax.experimental import pallas as pl
from jax.experimental.pallas import tpu as pltpu

N = 16
LOG_N = 4
B = 64
D = 512
H = 1024


def kernel(x, Win0, Wout0, Win1, Wout1, Win2, Wout2):
    def body(x_ref, win0_ref, wout0_ref, win1_ref, wout1_ref, win2_ref,
             wout2_ref, out_ref, xfull, ypart, sbuf, rbuf,
             ag_ss, ag_rs, rs_ss, rs_rs):
        i = lax.axis_index("i")

        barrier = pltpu.get_barrier_semaphore()
        for s in range(LOG_N):
            pl.semaphore_signal(barrier, inc=1, device_id=(i ^ (1 << s),),
                                device_id_type=pl.DeviceIdType.MESH)
        pl.semaphore_wait(barrier, LOG_N)

        def hyper_ag():
            for s in range(LOG_N):
                half = 1 << s
                rows = half * B
                partner = i ^ half
                my_c = (i // half) * half
                rdma = pltpu.make_async_remote_copy(
                    src_ref=xfull.at[pl.ds(my_c * B, rows)],
                    dst_ref=xfull.at[pl.ds(my_c * B, rows)],
                    send_sem=ag_ss.at[s],
                    recv_sem=ag_rs.at[s],
                    device_id=(partner,),
                    device_id_type=pl.DeviceIdType.MESH,
                )
                rdma.start()
                rdma.wait()

        def layer(win_ref, wout_ref):
            h1 = jnp.maximum(
                jnp.dot(xfull[...], win_ref[...].astype(jnp.bfloat16),
                        preferred_element_type=jnp.float32),
                0.0,
            ).astype(jnp.bfloat16)
            ypart[...] = jnp.dot(h1, wout_ref[...].astype(jnp.bfloat16),
                                 preferred_element_type=jnp.float32)

        def hyper_rs():
            for s in range(LOG_N):
                half = 8 >> s
                rows = half * B
                partner = i ^ half
                keep_c = (i // half) * half
                send_c = (partner // half) * half
                sbuf[s, :rows, :] = (
                    ypart[pl.ds(send_c * B, rows), :].astype(jnp.bfloat16)
                )
                rdma = pltpu.make_async_remote_copy(
                    src_ref=sbuf.at[s, pl.ds(0, rows)],
                    dst_ref=rbuf.at[s, pl.ds(0, rows)],
                    send_sem=rs_ss.at[s],
                    recv_sem=rs_rs.at[s],
                    device_id=(partner,),
                    device_id_type=pl.DeviceIdType.MESH,
                )
                rdma.start()
                rdma.wait()
                ypart[pl.ds(keep_c * B, rows), :] = (
                    ypart[pl.ds(keep_c * B, rows), :]
                    + rbuf[s, :rows, :].astype(jnp.float32)
                )

        xfull[pl.ds(i * B, B), :] = x_ref[...].astype(jnp.bfloat16)
        hyper_ag()

        for li, (win_ref, wout_ref) in enumerate(
            [(win0_ref, wout0_ref), (win1_ref, wout1_ref),
             (win2_ref, wout2_ref)]
        ):
            layer(win_ref, wout_ref)
            hyper_rs()
            if li < 2:
                xfull[pl.ds(i * B, B), :] = (
                    ypart[pl.ds(i * B, B), :].astype(jnp.bfloat16)
                )
                hyper_ag()

        out_ref[...] = ypart[pl.ds(i * B, B), :]

    vmem = pl.BlockSpec(memory_space=pltpu.VMEM)
    return pl.pallas_call(
        body,
        out_shape=jax.ShapeDtypeStruct((B, D), jnp.float32),
        in_specs=[vmem] * 7,
        out_specs=vmem,
        scratch_shapes=[
            pltpu.VMEM((N * B, D), jnp.bfloat16),
            pltpu.VMEM((N * B, D), jnp.float32),
            pltpu.VMEM((LOG_N, N * B // 2, D), jnp.bfloat16),
            pltpu.VMEM((LOG_N, N * B // 2, D), jnp.bfloat16),
            pltpu.SemaphoreType.DMA((LOG_N,)),
            pltpu.SemaphoreType.DMA((LOG_N,)),
            pltpu.SemaphoreType.DMA((LOG_N,)),
            pltpu.SemaphoreType.DMA((LOG_N,)),
        ],
        compiler_params=pltpu.CompilerParams(collective_id=0),
    )(x, Win0, Wout0, Win1, Wout1, Win2, Wout2)


# device time: 150231 ns/iter; 1.8833x vs baseline; 1.0584x over previous
import jax
import jax.numpy as jnp
from jax import lax
from jax.experimental import pallas as pl
from jax.experimental.pallas import tpu as pltpu

N = 16
B = 64
D = 512
H = 1024
NSEM = 5


def kernel(x, Win0, Wout0, Win1, Wout1, Win2, Wout2):
    def body(x_ref, win0_ref, wout0_ref, win1_ref, wout1_ref, win2_ref,
             wout2_ref, out_ref, xfull, ypart, sbuf, rbuf,
             ag_ss, ag_rs, rs_ss, rs_rs):
        i = lax.axis_index("i")

        barrier = pltpu.get_barrier_semaphore()
        for s in range(4):
            pl.semaphore_signal(barrier, inc=1, device_id=(i ^ (1 << s),),
                                device_id_type=pl.DeviceIdType.MESH)
        pl.semaphore_wait(barrier, 4)

        q2 = (i // 2) * 2
        q4 = (i // 4) * 4
        m8 = (i // 8) * 8
        o8 = m8 ^ 8
        s4 = q4 ^ 4
        s2 = q2 ^ 2
        s1 = i ^ 1

        def xchg(ref_at_fn, c, nchunks, partner, ss, rs_, slot):
            rdma = pltpu.make_async_remote_copy(
                src_ref=ref_at_fn(c, nchunks),
                dst_ref=ref_at_fn(c, nchunks),
                send_sem=ss.at[slot],
                recv_sem=rs_.at[slot],
                device_id=(partner,),
                device_id_type=pl.DeviceIdType.MESH,
            )
            rdma.start()
            return rdma

        def ag_xchg(c, nchunks, partner, slot):
            return xchg(lambda c_, n_: xfull.at[pl.ds(c_ * B, n_ * B)],
                        c, nchunks, partner, ag_ss, ag_rs, slot)

        def rs_send(c, nchunks, partner, slot):
            rows = nchunks * B
            sbuf[slot, :rows, :] = (
                ypart[pl.ds(c * B, rows), :].astype(jnp.bfloat16)
            )
            rdma = pltpu.make_async_remote_copy(
                src_ref=sbuf.at[slot, pl.ds(0, rows)],
                dst_ref=rbuf.at[slot, pl.ds(0, rows)],
                send_sem=rs_ss.at[slot],
                recv_sem=rs_rs.at[slot],
                device_id=(partner,),
                device_id_type=pl.DeviceIdType.MESH,
            )
            rdma.start()
            return rdma

        def rs_acc(c, nchunks, slot):
            rows = nchunks * B
            ypart[pl.ds(c * B, rows), :] = (
                ypart[pl.ds(c * B, rows), :]
                + rbuf[slot, :rows, :].astype(jnp.float32)
            )

        def run_layer(win_ref, wout_ref):
            win = win_ref[...].astype(jnp.bfloat16)
            wout = wout_ref[...].astype(jnp.bfloat16)

            def cmp(c, nchunks):
                rows = nchunks * B
                h1 = jnp.maximum(
                    jnp.dot(xfull[pl.ds(c * B, rows), :], win,
                            preferred_element_type=jnp.float32),
                    0.0,
                ).astype(jnp.bfloat16)
                ypart[pl.ds(c * B, rows), :] = jnp.dot(
                    h1, wout, preferred_element_type=jnp.float32)

            ag_xchg(i, 1, i ^ 1, 0).wait()
            ag_xchg(q2, 2, i ^ 2, 1).wait()
            ag2 = ag_xchg(q4, 4, i ^ 4, 2)
            cmp(q2, 2)
            cmp(s2, 2)
            ag2.wait()
            ag3a = ag_xchg(m8, 4, i ^ 8, 3)
            ag3b = ag_xchg(m8 + 4, 4, i ^ 8, 4)
            cmp(s4, 4)
            ag3a.wait()
            cmp(o8, 4)
            rs0a = rs_send(o8, 4, i ^ 8, 0)
            ag3b.wait()
            cmp(o8 + 4, 4)
            rs0b = rs_send(o8 + 4, 4, i ^ 8, 1)
            rs0a.wait()
            rs_acc(m8, 4, 0)
            rs0b.wait()
            rs_acc(m8 + 4, 4, 1)
            rs_send(s4, 4, i ^ 4, 2).wait()
            rs_acc(q4, 4, 2)
            rs_send(s2, 2, i ^ 2, 3).wait()
            rs_acc(q2, 2, 3)
            rs_send(s1, 1, i ^ 1, 4).wait()
            rs_acc(i, 1, 4)

        xfull[pl.ds(i * B, B), :] = x_ref[...].astype(jnp.bfloat16)
        run_layer(win0_ref, wout0_ref)
        xfull[pl.ds(i * B, B), :] = ypart[pl.ds(i * B, B), :].astype(jnp.bfloat16)
        run_layer(win1_ref, wout1_ref)
        xfull[pl.ds(i * B, B), :] = ypart[pl.ds(i * B, B), :].astype(jnp.bfloat16)
        run_layer(win2_ref, wout2_ref)
        out_ref[...] = ypart[pl.ds(i * B, B), :]

    vmem = pl.BlockSpec(memory_space=pltpu.VMEM)
    return pl.pallas_call(
        body,
        out_shape=jax.ShapeDtypeStruct((B, D), jnp.float32),
        in_specs=[vmem] * 7,
        out_specs=vmem,
        scratch_shapes=[
            pltpu.VMEM((N * B, D), jnp.bfloat16),
            pltpu.VMEM((N * B, D), jnp.float32),
            pltpu.VMEM((NSEM, 4 * B, D), jnp.bfloat16),
            pltpu.VMEM((NSEM, 4 * B, D), jnp.bfloat16),
            pltpu.SemaphoreType.DMA((NSEM,)),
            pltpu.SemaphoreType.DMA((NSEM,)),
            pltpu.SemaphoreType.DMA((NSEM,)),
            pltpu.SemaphoreType.DMA((NSEM,)),
        ],
        compiler_params=pltpu.CompilerParams(collective_id=0),
    )(x, Win0, Wout0, Win1, Wout1, Win2, Wout2)


# device time: 93851 ns/iter; 3.0146x vs baseline; 1.6007x over previous
import jax
import jax.numpy as jnp
from jax import lax
from jax.experimental import pallas as pl
from jax.experimental.pallas import tpu as pltpu

N = 16
B = 64
D = 512
H = 1024
HALF = N // 2


def kernel(x, Win0, Wout0, Win1, Wout1, Win2, Wout2):
    def body(x_ref, win0_ref, wout0_ref, win1_ref, wout1_ref, win2_ref,
             wout2_ref, out_ref, xfull, ypart, sbuf, rbuf,
             ag_ss, ag_rs, rs_ss, rs_rs):
        i = lax.axis_index("i")

        barrier = pltpu.get_barrier_semaphore()
        for d in range(1, N):
            pl.semaphore_signal(barrier, inc=1, device_id=((i + d) % N,),
                                device_id_type=pl.DeviceIdType.MESH)
        pl.semaphore_wait(barrier, N - 1)

        def ag_recv_waiter(plane, d):
            return pltpu.make_async_remote_copy(
                src_ref=xfull.at[plane, pl.ds((N - d) * B, B)],
                dst_ref=xfull.at[plane, pl.ds((N - d) * B, B)],
                send_sem=ag_ss.at[d],
                recv_sem=ag_rs.at[d],
                device_id=(i,),
                device_id_type=pl.DeviceIdType.MESH,
            )

        def run_layer(l, win_ref, wout_ref, last):
            plane = l % 2

            ag_rdmas = []
            for d in range(1, N):
                r = pltpu.make_async_remote_copy(
                    src_ref=xfull.at[plane, pl.ds(0, B)],
                    dst_ref=xfull.at[plane, pl.ds((N - d) * B, B)],
                    send_sem=ag_ss.at[d],
                    recv_sem=ag_rs.at[d],
                    device_id=((i + d) % N,),
                    device_id_type=pl.DeviceIdType.MESH,
                )
                r.start()
                ag_rdmas.append(r)

            win = win_ref[...].astype(jnp.bfloat16)
            wout = wout_ref[...].astype(jnp.bfloat16)

            rs_rdmas = []

            def compute_and_scatter(row0, rows, klo, khi):
                h1 = jnp.maximum(
                    jnp.dot(xfull[plane, pl.ds(row0, rows), :], win,
                            preferred_element_type=jnp.float32),
                    0.0,
                ).astype(jnp.bfloat16)
                ypart[pl.ds(row0, rows), :] = jnp.dot(
                    h1, wout, preferred_element_type=jnp.float32)
                for k in range(klo, khi):
                    sbuf[k, :, :] = (
                        ypart[pl.ds(k * B, B), :].astype(jnp.bfloat16))
                    r = pltpu.make_async_remote_copy(
                        src_ref=sbuf.at[k],
                        dst_ref=rbuf.at[N - k],
                        send_sem=rs_ss.at[k],
                        recv_sem=rs_rs.at[k],
                        device_id=((i + k) % N,),
                        device_id_type=pl.DeviceIdType.MESH,
                    )
                    r.start()
                    rs_rdmas.append(r)

            for q in (3, 2, 1, 0):
                for d in range(N - 4 * q - 3, min(N - 4 * q + 1, N)):
                    ag_recv_waiter(plane, d).wait_recv()
                compute_and_scatter(4 * q * B, 4 * B,
                                    max(4 * q, 1), 4 * q + 4)

            for d in range(1, N):
                pltpu.make_async_remote_copy(
                    src_ref=rbuf.at[N - d] if d < N else rbuf.at[1],
                    dst_ref=rbuf.at[N - d],
                    send_sem=rs_ss.at[d],
                    recv_sem=rs_rs.at[d],
                    device_id=(i,),
                    device_id_type=pl.DeviceIdType.MESH,
                ).wait_recv()
            acc = ypart[pl.ds(0, B), :] + jnp.sum(
                rbuf[1:N].astype(jnp.float32), axis=0)

            for r in ag_rdmas:
                r.wait_send()
            for r in rs_rdmas:
                r.wait_send()

            if last:
                out_ref[...] = acc
            else:
                xfull[(l + 1) % 2, pl.ds(0, B), :] = acc.astype(jnp.bfloat16)

        xfull[0, pl.ds(0, B), :] = x_ref[...].astype(jnp.bfloat16)
        run_layer(0, win0_ref, wout0_ref, False)
        run_layer(1, win1_ref, wout1_ref, False)
        run_layer(2, win2_ref, wout2_ref, True)

    vmem = pl.BlockSpec(memory_space=pltpu.VMEM)
    return pl.pallas_call(
        body,
        out_shape=jax.ShapeDtypeStruct((B, D), jnp.float32),
        in_specs=[vmem] * 7,
        out_specs=vmem,
        scratch_shapes=[
            pltpu.VMEM((2, N * B, D), jnp.bfloat16),
            pltpu.VMEM((N * B, D), jnp.float32),
            pltpu.VMEM((N, B, D), jnp.bfloat16),
            pltpu.VMEM((N, B, D), jnp.bfloat16),
            pltpu.SemaphoreType.DMA((N,)),
            pltpu.SemaphoreType.DMA((N,)),
            pltpu.SemaphoreType.DMA((N,)),
            pltpu.SemaphoreType.DMA((N,)),
        ],
        compiler_params=pltpu.CompilerParams(collective_id=0),
    )(x, Win0, Wout0, Win1, Wout1, Win2, Wout2)
